# Initial kernel scaffold; baseline (speedup 1.0000x reference)
import jax
import jax.numpy as jnp
from jax.experimental import pallas as pl


def kernel(sequence, segment_label, token_table, segment_table, pe):
    raise NotImplementedError("write your pallas kernel here")


# speedup vs baseline: 3.4372x; 3.4372x over previous
import functools

import jax
import jax.numpy as jnp
from jax import lax
from jax.experimental import pallas as pl
from jax.experimental.pallas import tpu as pltpu
from jax.experimental.pallas import tpu_sc as plsc

B, L, V, E = 1024, 512, 100000, 128

_NC, _NS = 2, 16
_NW = _NC * _NS
_ROWS = B * L
_RPW = _ROWS // _NW
_C = 256
_NCHUNK = _RPW // _C
_LANES = 16
_VPR = E // _LANES


def _body(seq_hbm, lab_hbm, tok_hbm, seg_hbm, pe_hbm, out_hbm,
          idx_v, lab_v, rows_v, pe_v, seg_v, sem):
    wid = lax.axis_index("s") * _NC + lax.axis_index("c")
    wbase = wid * _RPW

    pltpu.sync_copy(pe_hbm, pe_v)
    pltpu.sync_copy(seg_hbm, seg_v)
    lane = lax.iota(jnp.int32, _LANES)

    def chunk_step(t, carry):
        base = wbase + t * _C
        pe_off = (t % 2) * _C
        pltpu.sync_copy(seq_hbm.at[pl.ds(base, _C)], idx_v)
        pltpu.sync_copy(lab_hbm.at[pl.ds(base, _C)], lab_v)
        pltpu.async_copy(tok_hbm.at[idx_v], rows_v, sem).wait()

        def group_step(g, carry2):
            lab16 = lab_v[pl.ds(g * _LANES, _LANES)]
            for j in range(_LANES):
                lab_b = lab16.at[jnp.full((_LANES,), j, jnp.int32)].get(
                    mode="promise_in_bounds")
                row = g * _LANES + j
                for k in range(_VPR):
                    sl = pl.ds(k * _LANES, _LANES)
                    tok = rows_v[row, sl]
                    pv = pe_v[pe_off + row, sl]
                    sv = plsc.load_gather(seg_v, [lab_b * E + lane + k * _LANES])
                    rows_v[row, sl] = tok + pv + sv
            return carry2

        lax.fori_loop(0, _C // _LANES, group_step, 0, unroll=False)
        pltpu.sync_copy(rows_v, out_hbm.at[pl.ds(base, _C)])
        return carry

    lax.fori_loop(0, _NCHUNK, chunk_step, 0, unroll=False)


@jax.jit
def _embed(seq_flat, lab_flat, token_table, segment_table, pe2d):
    mesh = plsc.VectorSubcoreMesh(core_axis_name="c", subcore_axis_name="s")
    f = pl.kernel(
        _body,
        out_type=jax.ShapeDtypeStruct((_ROWS, E), jnp.float32),
        mesh=mesh,
        compiler_params=pltpu.CompilerParams(needs_layout_passes=False),
        scratch_types=[
            pltpu.VMEM((_C,), jnp.int32),
            pltpu.VMEM((_C,), jnp.int32),
            pltpu.VMEM((_C, E), jnp.float32),
            pltpu.VMEM((L, E), jnp.float32),
            pltpu.VMEM((3 * E,), jnp.float32),
            pltpu.SemaphoreType.DMA,
        ],
    )
    return f(seq_flat, lab_flat, token_table, segment_table, pe2d)


def kernel(sequence, segment_label, token_table, segment_table, pe):
    seq_flat = sequence.reshape(_ROWS).astype(jnp.int32)
    lab_flat = segment_label.reshape(_ROWS).astype(jnp.int32)
    pe2d = pe.reshape(L, E)
    out = _embed(seq_flat, lab_flat, token_table,
                 segment_table.reshape(3 * E), pe2d)
    return out.reshape(B, L, E)


# speedup vs baseline: 7.4410x; 2.1648x over previous
import functools

import jax
import jax.numpy as jnp
from jax import lax
from jax.experimental import pallas as pl
from jax.experimental.pallas import tpu as pltpu
from jax.experimental.pallas import tpu_sc as plsc

B, L, V, E = 1024, 512, 100000, 128

_NC, _NS = 2, 16
_NW = _NC * _NS
_ROWS = B * L
_RPW = _ROWS // _NW
_C = 256
_NCHUNK = _RPW // _C
_LANES = 16
_VPR = E // _LANES


def _body(seq_hbm, lab_hbm, tok_hbm, seg_hbm, pe_hbm, out_hbm,
          idx_v, lab_v, rows_v, pe_v, seg_v, sem):
    wid = lax.axis_index("s") * _NC + lax.axis_index("c")
    wbase = wid * _RPW

    pltpu.sync_copy(pe_hbm, pe_v)
    pltpu.sync_copy(seg_hbm, seg_v)
    lane = lax.iota(jnp.int32, _LANES)

    def chunk_step(t, carry):
        base = wbase + t * _C
        pe_off = (t % 2) * _C
        pltpu.sync_copy(seq_hbm.at[pl.ds(base, _C)], idx_v)
        pltpu.sync_copy(lab_hbm.at[pl.ds(base, _C)], lab_v)
        pltpu.async_copy(tok_hbm.at[idx_v], rows_v, sem).wait()

        seg1 = [seg_v[pl.ds(E + k * _LANES, _LANES)] for k in range(_VPR)]
        seg2 = [seg_v[pl.ds(2 * E + k * _LANES, _LANES)] for k in range(_VPR)]

        @plsc.parallel_loop(0, _C, unroll=4)
        def _(r):
            g = r // _LANES
            j = r % _LANES
            lab16 = lab_v[pl.ds(g * _LANES, _LANES)]
            labj = lab16.at[jnp.broadcast_to(j, (_LANES,))].get(
                mode="promise_in_bounds")
            m1 = labj == 1
            m2 = labj == 2
            for k in range(_VPR):
                sl = pl.ds(k * _LANES, _LANES)
                a = rows_v[r, sl] + pe_v[pe_off + r, sl]
                b = jnp.where(m1, seg1[k], 0.0) + jnp.where(m2, seg2[k], 0.0)
                rows_v[r, sl] = a + b
        pltpu.sync_copy(rows_v, out_hbm.at[pl.ds(base, _C)])
        return carry

    lax.fori_loop(0, _NCHUNK, chunk_step, 0, unroll=False)


@jax.jit
def _embed(seq_flat, lab_flat, token_table, segment_table, pe2d):
    mesh = plsc.VectorSubcoreMesh(core_axis_name="c", subcore_axis_name="s")
    f = pl.kernel(
        _body,
        out_type=jax.ShapeDtypeStruct((_ROWS, E), jnp.float32),
        mesh=mesh,
        compiler_params=pltpu.CompilerParams(needs_layout_passes=False),
        scratch_types=[
            pltpu.VMEM((_C,), jnp.int32),
            pltpu.VMEM((_C,), jnp.int32),
            pltpu.VMEM((_C, E), jnp.float32),
            pltpu.VMEM((L, E), jnp.float32),
            pltpu.VMEM((3 * E,), jnp.float32),
            pltpu.SemaphoreType.DMA,
        ],
    )
    return f(seq_flat, lab_flat, token_table, segment_table, pe2d)


def kernel(sequence, segment_label, token_table, segment_table, pe):
    seq_flat = sequence.reshape(_ROWS).astype(jnp.int32)
    lab_flat = segment_label.reshape(_ROWS).astype(jnp.int32)
    pe2d = pe.reshape(L, E)
    out = _embed(seq_flat, lab_flat, token_table,
                 segment_table.reshape(3 * E), pe2d)
    return out.reshape(B, L, E)


# speedup vs baseline: 12.9534x; 1.7408x over previous
import functools

import jax
import jax.numpy as jnp
from jax import lax
from jax.experimental import pallas as pl
from jax.experimental.pallas import tpu as pltpu
from jax.experimental.pallas import tpu_sc as plsc

B, L, V, E = 1024, 512, 100000, 128

_NC, _NS = 2, 16
_NW = _NC * _NS
_ROWS = B * L
_RPW = _ROWS // _NW
_C = 256
_NCH = L // _C * (B // _NW)
_BPW = B // _NW
_LANES = 16
_VPR = E // _LANES
_HALF = _BPW


def _body(seq_hbm, lab_hbm, tok_hbm, seg_hbm, pe_hbm, out_hbm,
          idx0, idx1, lab0, lab1, rows0, rows1, pe_v, seg_v,
          sg0, sg1, so0, so1, si0, si1):
    wid = lax.axis_index("s") * _NC + lax.axis_index("c")
    wbase = wid * _RPW

    pltpu.sync_copy(seg_hbm, seg_v)
    seg1 = [seg_v[pl.ds(E + k * _LANES, _LANES)] for k in range(_VPR)]
    seg2 = [seg_v[pl.ds(2 * E + k * _LANES, _LANES)] for k in range(_VPR)]

    idx_v = (idx0, idx1)
    lab_v = (lab0, lab1)
    rows_v = (rows0, rows1)
    sem_g = (sg0, sg1)
    sem_o = (so0, so1)
    sem_i = (si0, si1)

    def compute(rows, lab):
        @plsc.parallel_loop(0, _C, unroll=4)
        def _(r):
            g = r // _LANES
            j = r % _LANES
            lab16 = lab[pl.ds(g * _LANES, _LANES)]
            labj = lab16.at[jnp.broadcast_to(j, (_LANES,))].get(
                mode="promise_in_bounds")
            m1 = labj == 1
            m2 = labj == 2
            for k in range(_VPR):
                sl = pl.ds(k * _LANES, _LANES)
                a = rows[r, sl] + pe_v[r, sl]
                b = jnp.where(m1, seg1[k], 0.0) + jnp.where(m2, seg2[k], 0.0)
                rows[r, sl] = a + b

    for h in range(2):
        pltpu.sync_copy(pe_hbm.at[pl.ds(h * _C, _C)], pe_v)

        def base_of(c, h=h):
            return wbase + c * L + h * _C

        pltpu.sync_copy(seq_hbm.at[pl.ds(base_of(0), _C)], idx_v[0])
        pltpu.sync_copy(lab_hbm.at[pl.ds(base_of(0), _C)], lab_v[0])
        pltpu.async_copy(tok_hbm.at[idx_v[0]], rows_v[0], sem_g[0])
        pltpu.async_copy(seq_hbm.at[pl.ds(base_of(1), _C)], idx_v[1], sem_i[1])
        pltpu.async_copy(lab_hbm.at[pl.ds(base_of(1), _C)], lab_v[1], sem_i[1])

        def pair_step(i, carry, h=h):
            def base_of(c):
                return wbase + c * L + h * _C

            for p in range(2):
                c = i * 2 + p
                q = 1 - p
                pltpu.make_async_copy(tok_hbm.at[idx_v[p]], rows_v[p],
                                      sem_g[p]).wait()

                def start_next(first):
                    def go():
                        if not first:
                            pltpu.make_async_copy(
                                rows_v[q],
                                out_hbm.at[pl.ds(base_of(c - 1), _C)],
                                sem_o[q]).wait()
                        pltpu.make_async_copy(
                            seq_hbm.at[pl.ds(base_of(c + 1), _C)], idx_v[q],
                            sem_i[q]).wait()
                        pltpu.make_async_copy(
                            lab_hbm.at[pl.ds(base_of(c + 1), _C)], lab_v[q],
                            sem_i[q]).wait()
                        pltpu.async_copy(tok_hbm.at[idx_v[q]], rows_v[q],
                                         sem_g[q])
                    return go

                if p == 0:
                    pl.when(i == 0)(start_next(True))
                    pl.when(i > 0)(start_next(False))
                else:
                    pl.when(i < _BPW // 2 - 1)(start_next(False))

                def prefetch():
                    pltpu.async_copy(
                        seq_hbm.at[pl.ds(base_of(c + 2), _C)], idx_v[p],
                        sem_i[p])
                    pltpu.async_copy(
                        lab_hbm.at[pl.ds(base_of(c + 2), _C)], lab_v[p],
                        sem_i[p])

                compute(rows_v[p], lab_v[p])
                pl.when(i < _BPW // 2 - 1)(prefetch)
                pltpu.async_copy(rows_v[p],
                                 out_hbm.at[pl.ds(base_of(c), _C)], sem_o[p])
            return carry

        lax.fori_loop(0, _BPW // 2, pair_step, 0)
        pltpu.make_async_copy(rows_v[0],
                              out_hbm.at[pl.ds(base_of(_BPW - 2), _C)],
                              sem_o[0]).wait()
        pltpu.make_async_copy(rows_v[1],
                              out_hbm.at[pl.ds(base_of(_BPW - 1), _C)],
                              sem_o[1]).wait()


@jax.jit
def _embed(seq_flat, lab_flat, token_table, segment_table, pe2d):
    mesh = plsc.VectorSubcoreMesh(core_axis_name="c", subcore_axis_name="s")
    f = pl.kernel(
        _body,
        out_type=jax.ShapeDtypeStruct((_ROWS, E), jnp.float32),
        mesh=mesh,
        compiler_params=pltpu.CompilerParams(needs_layout_passes=False),
        scratch_types=[
            pltpu.VMEM((_C,), jnp.int32),
            pltpu.VMEM((_C,), jnp.int32),
            pltpu.VMEM((_C,), jnp.int32),
            pltpu.VMEM((_C,), jnp.int32),
            pltpu.VMEM((_C, E), jnp.float32),
            pltpu.VMEM((_C, E), jnp.float32),
            pltpu.VMEM((_C, E), jnp.float32),
            pltpu.VMEM((3 * E,), jnp.float32),
            pltpu.SemaphoreType.DMA,
            pltpu.SemaphoreType.DMA,
            pltpu.SemaphoreType.DMA,
            pltpu.SemaphoreType.DMA,
            pltpu.SemaphoreType.DMA,
            pltpu.SemaphoreType.DMA,
        ],
    )
    return f(seq_flat, lab_flat, token_table, segment_table, pe2d)


def kernel(sequence, segment_label, token_table, segment_table, pe):
    seq_flat = sequence.reshape(_ROWS).astype(jnp.int32)
    lab_flat = segment_label.reshape(_ROWS).astype(jnp.int32)
    pe2d = pe.reshape(L, E)
    out = _embed(seq_flat, lab_flat, token_table,
                 segment_table.reshape(3 * E), pe2d)
    return out.reshape(B, L, E)
